# baseline (speedup 1.0000x reference)
import functools

import jax
import jax.numpy as jnp
from jax import lax
from jax.experimental import pallas as pl
from jax.experimental.pallas import tpu as pltpu

HIDDEN_DIM = 2048
DELTA = 1.0 / HIDDEN_DIM
GAMMA = 0.05
LAMBDA_KL = 0.01
BETA = 0.05

ROWS = 4 * 8192
BLOCK_ROWS = 512
NSTEPS = ROWS // BLOCK_ROWS


def _reduce_body(x_ref, e_ref, n_ref,
                 ne_ref, aux_ref, msk_ref, val_ref, nmask_ref,
                 acc_ref, cnt_ref):
    i = pl.program_id(0)
    xb = x_ref[...]
    relu = jnp.maximum(xb, 0.0)
    psum = jnp.sum(relu, axis=0, keepdims=True)
    pcnt = jnp.sum((xb > 0.0).astype(jnp.float32), axis=0, keepdims=True)

    @pl.when(i == 0)
    def _():
        acc_ref[...] = psum
        cnt_ref[...] = pcnt

    @pl.when(i > 0)
    def _():
        acc_ref[...] += psum
        cnt_ref[...] += pcnt

    @pl.when(i == NSTEPS - 1)
    def _():
        colmean = acc_ref[...] * (1.0 / ROWS)
        e = e_ref[...]
        ne = e + DELTA + n_ref[...] - GAMMA * colmean
        rho = jnp.sum(cnt_ref[...]) * (1.0 / (ROWS * HIDDEN_DIM))
        rho = jnp.clip(rho, 1e-05, 1.0 - 1e-05)
        kl = LAMBDA_KL * (rho * jnp.log(rho / BETA)
                          + (1.0 - rho) * jnp.log((1.0 - rho) / (1.0 - BETA)))
        high = ne > 1.0
        low = ne < -1.0
        pen = (0.01 * jnp.sum(jnp.where(high, jnp.abs(ne) - 1.0, 0.0))
               + 0.01 * jnp.sum(jnp.where(low, jnp.abs(ne) - 1.0, 0.0)))
        aux_ref[0, 0] = kl + pen
        fire = ne >= 2.0
        shut = ne <= -2.0
        ne_ref[...] = jnp.where(shut, -2.0, ne)
        m = jnp.logical_or(fire, shut)
        msk_ref[...] = m.astype(jnp.float32)
        val_ref[...] = jnp.where(shut, e + 2.0, 2.0)
        nmask_ref[0, 0] = jnp.sum(m.astype(jnp.int32))


def _blend_body(x_ref, msk_ref, val_ref, h_ref):
    h_ref[...] = jnp.where(msk_ref[...] > 0.5, val_ref[...], x_ref[...])


def _blend(xf, msk, val):
    return pl.pallas_call(
        _blend_body,
        grid=(NSTEPS,),
        in_specs=[
            pl.BlockSpec((BLOCK_ROWS, HIDDEN_DIM), lambda i: (i, 0)),
            pl.BlockSpec((1, HIDDEN_DIM), lambda i: (0, 0)),
            pl.BlockSpec((1, HIDDEN_DIM), lambda i: (0, 0)),
        ],
        out_specs=pl.BlockSpec((BLOCK_ROWS, HIDDEN_DIM), lambda i: (i, 0)),
        out_shape=jax.ShapeDtypeStruct((ROWS, HIDDEN_DIM), jnp.float32),
        compiler_params=pltpu.CompilerParams(
            dimension_semantics=("parallel",),
        ),
    )(xf, msk, val)


@jax.jit
def kernel(x, energy, noise):
    xf = x.reshape(ROWS, HIDDEN_DIM)
    e2 = energy.reshape(1, HIDDEN_DIM)
    n2 = noise.reshape(1, HIDDEN_DIM)

    ne, aux, msk, val, nmask = pl.pallas_call(
        _reduce_body,
        grid=(NSTEPS,),
        in_specs=[
            pl.BlockSpec((BLOCK_ROWS, HIDDEN_DIM), lambda i: (i, 0)),
            pl.BlockSpec((1, HIDDEN_DIM), lambda i: (0, 0)),
            pl.BlockSpec((1, HIDDEN_DIM), lambda i: (0, 0)),
        ],
        out_specs=[
            pl.BlockSpec((1, HIDDEN_DIM), lambda i: (0, 0)),
            pl.BlockSpec((1, 1), lambda i: (0, 0), memory_space=pltpu.SMEM),
            pl.BlockSpec((1, HIDDEN_DIM), lambda i: (0, 0)),
            pl.BlockSpec((1, HIDDEN_DIM), lambda i: (0, 0)),
            pl.BlockSpec((1, 1), lambda i: (0, 0), memory_space=pltpu.SMEM),
        ],
        out_shape=[
            jax.ShapeDtypeStruct((1, HIDDEN_DIM), jnp.float32),
            jax.ShapeDtypeStruct((1, 1), jnp.float32),
            jax.ShapeDtypeStruct((1, HIDDEN_DIM), jnp.float32),
            jax.ShapeDtypeStruct((1, HIDDEN_DIM), jnp.float32),
            jax.ShapeDtypeStruct((1, 1), jnp.int32),
        ],
        scratch_shapes=[
            pltpu.VMEM((1, HIDDEN_DIM), jnp.float32),
            pltpu.VMEM((1, HIDDEN_DIM), jnp.float32),
        ],
        compiler_params=pltpu.CompilerParams(
            dimension_semantics=("arbitrary",),
        ),
    )(xf, e2, n2)

    h = lax.cond(
        nmask[0, 0] > 0,
        lambda: _blend(xf, msk, val),
        lambda: xf,
    )

    return (h.reshape(x.shape), ne.reshape(HIDDEN_DIM), aux[0, 0])
